# Initial kernel scaffold; baseline (speedup 1.0000x reference)
import jax
import jax.numpy as jnp
from jax.experimental import pallas as pl


def kernel(x, conv1_w, bn1_g, bn1_b, conv2_w, bn2_g, bn2_b, conv3_w, bn3_g, bn3_b, conv4_w, bn4_g, bn4_b, conv5_w, bn5_g, bn5_b, fc1_w, bn6_g, bn6_b, fc2_w, fc2_b, bn7_g, bn7_b, fc3_w, fc3_b):
    raise NotImplementedError("write your pallas kernel here")


# speedup vs baseline: 10.0280x; 10.0280x over previous
import functools

import jax
import jax.numpy as jnp
from jax import lax
from jax.experimental import pallas as pl
from jax.experimental.pallas import tpu as pltpu
from jax.experimental.pallas import tpu_sc as plsc

B = 16
N = 1024
BN = B * N
KNN = 20
PC = 128
NW = 32
RPW = BN * KNN // NW
RPCH = 80
NCH = RPW // RPCH
NBUF = 4
PT = 256
CNT = float(BN * KNN)
EPS = 1e-5


def _lrelu(x):
    return jnp.where(x >= 0, x, 0.2 * x)


def _dot_nt(a, b):
    return lax.dot_general(a, b, (((1,), (1,)), ((), ())),
                           preferred_element_type=jnp.float32)


def _graph_kernel(xt_ref, gidx_ref, xp_ref, pd_ref):
    b = pl.program_id(0)
    xt = xt_ref[...]
    c = xt.shape[1]
    if c < PC:
        xp_ref[...] = jnp.concatenate(
            [xt, jnp.zeros((N, PC - c), jnp.float32)], axis=1)
    else:
        xp_ref[...] = xt
    xb = xt.astype(jnp.bfloat16)
    inner = _dot_nt(xb, xb)
    sq = jnp.sum(xt * xt, axis=1)
    pd_ref[...] = 2.0 * inner - sq[:, None] - sq[None, :]
    iota = lax.broadcasted_iota(jnp.int32, (N, N), 1)
    cols = []
    for _ in range(KNN):
        pd = pd_ref[...]
        m = jnp.max(pd, axis=1)
        am = jnp.min(jnp.where(pd == m[:, None], iota, N), axis=1)
        cols.append(am[:, None])
        pd_ref[...] = jnp.where(iota == am[:, None], -jnp.inf, pd)
    gidx_ref[...] = jnp.concatenate(cols, axis=1) + b * N


def _graph_build(xt):
    c = xt.shape[-1]
    return pl.pallas_call(
        _graph_kernel,
        grid=(B,),
        in_specs=[pl.BlockSpec((N, c), lambda b: (b, 0))],
        out_specs=[
            pl.BlockSpec((N, KNN), lambda b: (b, 0)),
            pl.BlockSpec((N, PC), lambda b: (b, 0)),
        ],
        out_shape=[
            jax.ShapeDtypeStruct((BN, KNN), jnp.int32),
            jax.ShapeDtypeStruct((BN, PC), jnp.float32),
        ],
        scratch_shapes=[pltpu.VMEM((N, N), jnp.float32)],
    )(xt)


def _make_sc_gather():
    mesh = plsc.VectorSubcoreMesh(core_axis_name="c", subcore_axis_name="s")

    @functools.partial(
        pl.kernel,
        mesh=mesh,
        out_type=jax.ShapeDtypeStruct((BN * KNN, PC), jnp.float32),
        scratch_types=[
            pltpu.VMEM((RPW,), jnp.int32),
            pltpu.VMEM((NBUF, RPCH, PC), jnp.float32),
            pltpu.SemaphoreType.DMA,
            pltpu.SemaphoreType.DMA,
            pltpu.SemaphoreType.DMA,
            pltpu.SemaphoreType.DMA,
            pltpu.SemaphoreType.DMA,
            pltpu.SemaphoreType.DMA,
            pltpu.SemaphoreType.DMA,
            pltpu.SemaphoreType.DMA,
        ],
    )
    def sck(xp_hbm, gidx_hbm, out_hbm, idx_v, rows_v, g0, g1, g2, g3,
            w0, w1, w2, w3):
        cid = lax.axis_index("c")
        sid = lax.axis_index("s")
        wid = sid * 2 + cid
        rbase = wid * RPW
        pltpu.sync_copy(gidx_hbm.at[pl.ds(rbase, RPW)], idx_v)
        gsem = (g0, g1, g2, g3)
        wsem = (w0, w1, w2, w3)

        def issue_gather(g):
            pltpu.async_copy(
                xp_hbm.at[idx_v.at[pl.ds(g * RPCH, RPCH)]],
                rows_v.at[g % NBUF], gsem[g % NBUF])

        def wait_gather(g):
            pltpu.make_async_copy(
                xp_hbm.at[pl.ds(0, RPCH)], rows_v.at[g % NBUF],
                gsem[g % NBUF]).wait()

        def issue_write(g):
            pltpu.async_copy(
                rows_v.at[g % NBUF],
                out_hbm.at[pl.ds(rbase + g * RPCH, RPCH)], wsem[g % NBUF])

        def wait_write(g):
            pltpu.make_async_copy(
                rows_v.at[g % NBUF], out_hbm.at[pl.ds(rbase, RPCH)],
                wsem[g % NBUF]).wait()

        issue_gather(0)
        issue_gather(1)
        for g in range(NCH):
            wait_gather(g)
            issue_write(g)
            if g + 2 < NCH:
                if g >= 2:
                    wait_write(g - 2)
                issue_gather(g + 2)
        wait_write(NCH - 2)
        wait_write(NCH - 1)

    return sck


_SC_GATHER = None


def _gather_rows(xp, gidx_flat):
    global _SC_GATHER
    if _SC_GATHER is None:
        _SC_GATHER = _make_sc_gather()
    return _SC_GATHER(xp, gidx_flat)


def _pair_kernel(c, o, gx_ref, xt_ref, w_ref, agg_ref):
    xn = xt_ref[...]
    gx = gx_ref[...][:, :c]
    xr = jnp.broadcast_to(xn[:, None, :], (PT, KNN, c)).reshape(PT * KNN, c)
    f = jnp.concatenate([(gx - xr).astype(jnp.bfloat16),
                         xr.astype(jnp.bfloat16)], axis=1)
    w = w_ref[...].astype(jnp.bfloat16)
    h = _dot_nt(f, w)
    h3 = h.reshape(PT, KNN, o)
    m = h3[:, 0, :]
    s = m
    q = m * m
    for j in range(1, KNN):
        v = h3[:, j, :]
        m = jnp.maximum(m, v)
        s = s + v
        q = q + v * v
    agg_ref[...] = jnp.concatenate([m, s, q], axis=1)


def _pair_reduce(gx, xt, w):
    o, c2 = w.shape
    c = c2 // 2
    return pl.pallas_call(
        functools.partial(_pair_kernel, c, o),
        grid=(BN // PT,),
        in_specs=[
            pl.BlockSpec((PT * KNN, PC), lambda i: (i, 0)),
            pl.BlockSpec((PT, c), lambda i: (i, 0)),
            pl.BlockSpec((o, c2), lambda i: (0, 0)),
        ],
        out_specs=pl.BlockSpec((PT, 3 * o), lambda i: (i, 0)),
        out_shape=jax.ShapeDtypeStruct((BN, 3 * o), jnp.float32),
    )(gx, xt, w)


def _stats_kernel(o, agg_ref, st_ref):
    i = pl.program_id(0)
    agg = agg_ref[...]
    blk = jnp.concatenate(
        [jnp.sum(agg[:, o:2 * o], axis=0)[None, :],
         jnp.sum(agg[:, 2 * o:3 * o], axis=0)[None, :],
         jnp.zeros((6, o), jnp.float32)], axis=0)

    @pl.when(i == 0)
    def _():
        st_ref[...] = blk

    @pl.when(i > 0)
    def _():
        st_ref[...] += blk


def _apply_kernel(o, agg_ref, st_ref, g_ref, b_ref, o_ref):
    st = st_ref[...]
    mean = st[0] / CNT
    var = st[1] / CNT - mean * mean
    den = jnp.sqrt(var + EPS)
    xh = (agg_ref[:, :o] - mean[None, :]) / den[None, :]
    o_ref[...] = _lrelu(xh * g_ref[0][None, :] + b_ref[0][None, :])


def _edge_block(xt, w, g, b):
    o = w.shape[0]
    gidx, xp = _graph_build(xt)
    gx = _gather_rows(xp, gidx.reshape(BN * KNN))
    agg = _pair_reduce(gx, xt, w)
    stats = pl.pallas_call(
        functools.partial(_stats_kernel, o),
        grid=(B,),
        in_specs=[pl.BlockSpec((N, 3 * o), lambda i: (i, 0))],
        out_specs=pl.BlockSpec((8, o), lambda i: (0, 0)),
        out_shape=jax.ShapeDtypeStruct((8, o), jnp.float32),
    )(agg)
    return pl.pallas_call(
        functools.partial(_apply_kernel, o),
        grid=(B,),
        in_specs=[
            pl.BlockSpec((N, 3 * o), lambda i: (i, 0)),
            pl.BlockSpec((8, o), lambda i: (0, 0)),
            pl.BlockSpec((1, o), lambda i: (0, 0)),
            pl.BlockSpec((1, o), lambda i: (0, 0)),
        ],
        out_specs=pl.BlockSpec((N, o), lambda i: (i, 0)),
        out_shape=jax.ShapeDtypeStruct((BN, o), jnp.float32),
    )(agg, stats, g.reshape(1, o), b.reshape(1, o))


_XCH = (64, 64, 128, 256)
_EMB = 1024


def _conv5_rows(x_refs, w_ref):
    w5 = w_ref[...].astype(jnp.bfloat16)
    y = jnp.zeros((N, _EMB), jnp.float32)
    off = 0
    for xr, ci in zip(x_refs, _XCH):
        y = y + _dot_nt(xr[...].astype(jnp.bfloat16), w5[:, off:off + ci])
        off += ci
    return y


def _conv5_stats_kernel(x0, x1, x2, x3, w_ref, st_ref):
    i = pl.program_id(0)
    y = _conv5_rows((x0, x1, x2, x3), w_ref)
    blk = jnp.concatenate(
        [jnp.sum(y, axis=0)[None, :], jnp.sum(y * y, axis=0)[None, :],
         jnp.zeros((6, _EMB), jnp.float32)], axis=0)

    @pl.when(i == 0)
    def _():
        st_ref[...] = blk

    @pl.when(i > 0)
    def _():
        st_ref[...] += blk


def _conv5_apply_kernel(x0, x1, x2, x3, w_ref, st_ref, g_ref, b_ref,
                        xm_ref, xa_ref):
    y = _conv5_rows((x0, x1, x2, x3), w_ref)
    st = st_ref[...]
    mean = st[0] / float(BN)
    var = st[1] / float(BN) - mean * mean
    den = jnp.sqrt(var + EPS)
    xh = (y - mean[None, :]) / den[None, :]
    z = _lrelu(xh * g_ref[0][None, :] + b_ref[0][None, :])
    xm_ref[0, 0, :] = jnp.max(z, axis=0)
    xa_ref[0, 0, :] = jnp.sum(z, axis=0) * (1.0 / N)


def _head_kernel(xm_ref, xa_ref, fc1_ref, g6_ref, b6_ref, fc2_ref, fb2_ref,
                 g7_ref, b7_ref, fc3_ref, fb3_ref, o_ref):
    fc1 = fc1_ref[...]
    h = (_dot_nt(xm_ref[...], fc1[:, :_EMB]) +
         _dot_nt(xa_ref[...], fc1[:, _EMB:]))

    def bn0(x, g, bb):
        mean = jnp.mean(x, axis=0)
        var = jnp.mean((x - mean[None, :]) ** 2, axis=0)
        xh = (x - mean[None, :]) / jnp.sqrt(var + EPS)[None, :]
        return xh * g[0][None, :] + bb[0][None, :]

    h = _lrelu(bn0(h, g6_ref, b6_ref))
    h = _dot_nt(h, fc2_ref[...]) + fb2_ref[0][None, :]
    h = _lrelu(bn0(h, g7_ref, b7_ref))
    o_ref[...] = _dot_nt(h, fc3_ref[...]) + fb3_ref[0][None, :]


def kernel(x, conv1_w, bn1_g, bn1_b, conv2_w, bn2_g, bn2_b, conv3_w, bn3_g,
           bn3_b, conv4_w, bn4_g, bn4_b, conv5_w, bn5_g, bn5_b, fc1_w, bn6_g,
           bn6_b, fc2_w, fc2_b, bn7_g, bn7_b, fc3_w, fc3_b):
    x0 = _edge_block(x.reshape(BN, 3), conv1_w, bn1_g, bn1_b)
    x1 = _edge_block(x0, conv2_w, bn2_g, bn2_b)
    x2 = _edge_block(x1, conv3_w, bn3_g, bn3_b)
    x3 = _edge_block(x2, conv4_w, bn4_g, bn4_b)

    xspecs = [pl.BlockSpec((N, ci), lambda i: (i, 0)) for ci in _XCH]
    wspec = pl.BlockSpec((_EMB, 512), lambda i: (0, 0))
    stats5 = pl.pallas_call(
        _conv5_stats_kernel,
        grid=(B,),
        in_specs=xspecs + [wspec],
        out_specs=pl.BlockSpec((8, _EMB), lambda i: (0, 0)),
        out_shape=jax.ShapeDtypeStruct((8, _EMB), jnp.float32),
    )(x0, x1, x2, x3, conv5_w)
    xm, xa = pl.pallas_call(
        _conv5_apply_kernel,
        grid=(B,),
        in_specs=xspecs + [
            wspec,
            pl.BlockSpec((8, _EMB), lambda i: (0, 0)),
            pl.BlockSpec((1, _EMB), lambda i: (0, 0)),
            pl.BlockSpec((1, _EMB), lambda i: (0, 0)),
        ],
        out_specs=[
            pl.BlockSpec((1, 1, _EMB), lambda i: (i, 0, 0)),
            pl.BlockSpec((1, 1, _EMB), lambda i: (i, 0, 0)),
        ],
        out_shape=[
            jax.ShapeDtypeStruct((B, 1, _EMB), jnp.float32),
            jax.ShapeDtypeStruct((B, 1, _EMB), jnp.float32),
        ],
    )(x0, x1, x2, x3, conv5_w, stats5, bn5_g.reshape(1, _EMB),
      bn5_b.reshape(1, _EMB))
    xm = xm.reshape(B, _EMB)
    xa = xa.reshape(B, _EMB)

    return pl.pallas_call(
        _head_kernel,
        out_shape=jax.ShapeDtypeStruct((B, 40), jnp.float32),
    )(xm, xa, fc1_w, bn6_g.reshape(1, 512), bn6_b.reshape(1, 512), fc2_w,
      fc2_b.reshape(1, 256), bn7_g.reshape(1, 256), bn7_b.reshape(1, 256),
      fc3_w, fc3_b.reshape(1, 40))
